# baseline (speedup 1.0000x reference)
import functools

import jax
import jax.numpy as jnp
from jax import lax
from jax.experimental import pallas as pl
from jax.experimental.pallas import tpu as pltpu, tpu_sc as plsc

_B, _P, _D = 256, 1024, 96
_ROWS_PER_BATCH = _D
_CHUNK = 8
_NW = 32
_BATCH_PER_W = _B // _NW
_CHUNKS_PER_BATCH = _ROWS_PER_BATCH // _CHUNK
_NCHUNK = _BATCH_PER_W * _CHUNKS_PER_BATCH


def _add_block(buf, slot, t_v, c):
    for j in range(_CHUNK):
        for k in range(1024 // 16):
            sl = pl.ds(k * 16, 16)
            buf[slot, j, sl] = buf[slot, j, sl] + t_v[c * _CHUNK + j, sl]


def _sc_kernel(x_hbm, t_hbm, o_hbm, t_v, buf, tsem, insems, outsems):
    nc = 2
    wid = lax.axis_index("s") * nc + lax.axis_index("c")
    row0 = wid * _BATCH_PER_W * _ROWS_PER_BATCH

    tcp = pltpu.make_async_copy(t_hbm, t_v, tsem)
    tcp.start()
    tcp.wait()

    def in_copy(i, slot):
        return pltpu.make_async_copy(
            x_hbm.at[pl.ds(row0 + i * _CHUNK, _CHUNK)],
            buf.at[slot],
            insems.at[slot],
        )

    def out_copy(i, slot):
        return pltpu.make_async_copy(
            buf.at[slot],
            o_hbm.at[pl.ds(row0 + i * _CHUNK, _CHUNK)],
            outsems.at[slot],
        )

    in_copy(0, 0).start()
    in_copy(1, 1).start()

    def step(i, slot):
        c = lax.rem(i, _CHUNKS_PER_BATCH)
        in_copy(i, slot).wait()

        @pl.when(i >= 2)
        def _():
            out_copy(i - 2, slot).wait()

        _add_block(buf, slot, t_v, c)
        out_copy(i, slot).start()

        @pl.when(i + 2 < _NCHUNK)
        def _():
            in_copy(i + 2, slot).start()

    def body(t, carry):
        step(2 * t, 0)
        step(2 * t + 1, 1)
        return carry

    lax.fori_loop(0, _NCHUNK // 2, body, 0)
    out_copy(_NCHUNK - 2, 0).wait()
    out_copy(_NCHUNK - 1, 1).wait()


def kernel(encoded_patches, pos_table):
    B, P, D = encoded_patches.shape
    xt = jnp.swapaxes(encoded_patches, 1, 2)
    x2d = xt.reshape(B * D, P)
    t2d = pos_table.T

    mesh = plsc.VectorSubcoreMesh(core_axis_name="c", subcore_axis_name="s")
    run = functools.partial(
        pl.kernel,
        mesh=mesh,
        out_type=jax.ShapeDtypeStruct((B * D, P), jnp.float32),
        scratch_types=[
            pltpu.VMEM((_ROWS_PER_BATCH, _P), jnp.float32),
            pltpu.VMEM((2, _CHUNK, _P), jnp.float32),
            pltpu.SemaphoreType.DMA,
            pltpu.SemaphoreType.DMA((2,)),
            pltpu.SemaphoreType.DMA((2,)),
        ],
    )(_sc_kernel)
    out2d = run(x2d, t2d)
    return jnp.swapaxes(out2d.reshape(B, D, P), 1, 2)


# speedup vs baseline: 2.7677x; 2.7677x over previous
import functools

import jax
import jax.numpy as jnp
from jax import lax
from jax.experimental import pallas as pl
from jax.experimental.pallas import tpu as pltpu, tpu_sc as plsc

_B, _P, _D = 256, 1024, 96
_ROWS_PER_BATCH = _D
_CHUNK = 8
_NW = 32
_BATCH_PER_W = _B // _NW
_CHUNKS_PER_BATCH = _ROWS_PER_BATCH // _CHUNK
_NCHUNK = _BATCH_PER_W * _CHUNKS_PER_BATCH


_VPR = _P // 16


def _add_block(buf, slot, t_v, c):
    base = c * _CHUNK

    @plsc.parallel_loop(0, _CHUNK * _VPR, unroll=8)
    def _(k):
        j = k // _VPR
        sl = pl.ds((k % _VPR) * 16, 16)
        buf[slot, j, sl] = buf[slot, j, sl] + t_v[base + j, sl]


def _sc_kernel(x_hbm, t_hbm, o_hbm, t_v, buf, tsem, insems, outsems):
    nc = 2
    wid = lax.axis_index("s") * nc + lax.axis_index("c")
    row0 = wid * _BATCH_PER_W * _ROWS_PER_BATCH

    tcp = pltpu.make_async_copy(t_hbm, t_v, tsem)
    tcp.start()
    tcp.wait()

    def in_copy(i, slot):
        return pltpu.make_async_copy(
            x_hbm.at[pl.ds(row0 + i * _CHUNK, _CHUNK)],
            buf.at[slot],
            insems.at[slot],
        )

    def out_copy(i, slot):
        return pltpu.make_async_copy(
            buf.at[slot],
            o_hbm.at[pl.ds(row0 + i * _CHUNK, _CHUNK)],
            outsems.at[slot],
        )

    in_copy(0, 0).start()
    in_copy(1, 1).start()

    def step(i, slot):
        c = lax.rem(i, _CHUNKS_PER_BATCH)
        in_copy(i, slot).wait()

        @pl.when(i >= 2)
        def _():
            out_copy(i - 2, slot).wait()

        _add_block(buf, slot, t_v, c)
        out_copy(i, slot).start()

        @pl.when(i + 2 < _NCHUNK)
        def _():
            in_copy(i + 2, slot).start()

    def body(t, carry):
        step(2 * t, 0)
        step(2 * t + 1, 1)
        return carry

    lax.fori_loop(0, _NCHUNK // 2, body, 0)
    out_copy(_NCHUNK - 2, 0).wait()
    out_copy(_NCHUNK - 1, 1).wait()


def kernel(encoded_patches, pos_table):
    B, P, D = encoded_patches.shape
    xt = jnp.swapaxes(encoded_patches, 1, 2)
    x2d = xt.reshape(B * D, P)
    t2d = pos_table.T

    mesh = plsc.VectorSubcoreMesh(core_axis_name="c", subcore_axis_name="s")
    run = functools.partial(
        pl.kernel,
        mesh=mesh,
        out_type=jax.ShapeDtypeStruct((B * D, P), jnp.float32),
        scratch_types=[
            pltpu.VMEM((_ROWS_PER_BATCH, _P), jnp.float32),
            pltpu.VMEM((2, _CHUNK, _P), jnp.float32),
            pltpu.SemaphoreType.DMA,
            pltpu.SemaphoreType.DMA((2,)),
            pltpu.SemaphoreType.DMA((2,)),
        ],
    )(_sc_kernel)
    out2d = run(x2d, t2d)
    return jnp.swapaxes(out2d.reshape(B, D, P), 1, 2)


# speedup vs baseline: 2.9520x; 1.0666x over previous
import functools

import jax
import jax.numpy as jnp
from jax import lax
from jax.experimental import pallas as pl
from jax.experimental.pallas import tpu as pltpu, tpu_sc as plsc

_B, _P, _D = 256, 1024, 96
_ROWS_PER_BATCH = _D
_CHUNK = 16
_NW = 32
_BATCH_PER_W = _B // _NW
_NPHASE = _ROWS_PER_BATCH // _CHUNK
_NSLOT = 4
_VPR = _P // 16


def _add_block(buf, slot, tbuf, tc):
    @plsc.parallel_loop(0, _CHUNK * _VPR, unroll=8)
    def _(k):
        j = k // _VPR
        sl = pl.ds((k % _VPR) * 16, 16)
        buf[slot, j, sl] = buf[slot, j, sl] + tbuf[tc, j, sl]


def _sc_kernel(x_hbm, t_hbm, o_hbm, tbuf, buf, tsems, insems, outsems):
    nc = 2
    wid = lax.axis_index("s") * nc + lax.axis_index("c")
    row0 = wid * _BATCH_PER_W * _ROWS_PER_BATCH

    def t_copy(c, tc):
        return pltpu.make_async_copy(
            t_hbm.at[pl.ds(c * _CHUNK, _CHUNK)], tbuf.at[tc], tsems.at[tc]
        )

    def in_copy(c, b, slot):
        rows = pl.ds(row0 + b * _ROWS_PER_BATCH + c * _CHUNK, _CHUNK)
        return pltpu.make_async_copy(x_hbm.at[rows], buf.at[slot], insems.at[slot])

    def out_copy(c, b, slot):
        rows = pl.ds(row0 + b * _ROWS_PER_BATCH + c * _CHUNK, _CHUNK)
        return pltpu.make_async_copy(buf.at[slot], o_hbm.at[rows], outsems.at[slot])

    t_copy(0, 0).start()
    t_copy(1, 1).start()
    in_copy(0, 0, 0).start()
    in_copy(0, 1, 1).start()

    def phase(c, carry):
        tc = lax.rem(c, 2)
        for b in range(_BATCH_PER_W):
            slot = b % _NSLOT
            in_copy(c, b, slot).wait()
            if b == 0:
                t_copy(c, tc).wait()
            _add_block(buf, slot, tbuf, tc)
            out_copy(c, b, slot).start()
            if b >= 2:
                out_copy(c, b - 2, (b - 2) % _NSLOT).wait()
                if b + 2 < _BATCH_PER_W:
                    in_copy(c, b + 2, (b + 2) % _NSLOT).start()
                else:
                    @pl.when(c < _NPHASE - 1)
                    def _():
                        in_copy(c + 1, b + 2 - _BATCH_PER_W, (b + 2) % _NSLOT).start()
            else:
                @pl.when(c > 0)
                def _():
                    out_copy(c - 1, b + _BATCH_PER_W - 2, (b + 2) % _NSLOT).wait()
                    in_copy(c, b + 2, (b + 2) % _NSLOT).start()

                @pl.when(c == 0)
                def _():
                    in_copy(c, b + 2, (b + 2) % _NSLOT).start()
            if b == _BATCH_PER_W - 1:
                @pl.when(c < _NPHASE - 2)
                def _():
                    t_copy(c + 2, tc).start()
        return carry

    lax.fori_loop(0, _NPHASE, phase, 0)
    out_copy(_NPHASE - 1, _BATCH_PER_W - 2, (_BATCH_PER_W - 2) % _NSLOT).wait()
    out_copy(_NPHASE - 1, _BATCH_PER_W - 1, (_BATCH_PER_W - 1) % _NSLOT).wait()


def kernel(encoded_patches, pos_table):
    B, P, D = encoded_patches.shape
    xt = jnp.swapaxes(encoded_patches, 1, 2)
    x2d = xt.reshape(B * D, P)
    t2d = pos_table.T

    mesh = plsc.VectorSubcoreMesh(core_axis_name="c", subcore_axis_name="s")
    run = functools.partial(
        pl.kernel,
        mesh=mesh,
        out_type=jax.ShapeDtypeStruct((B * D, P), jnp.float32),
        scratch_types=[
            pltpu.VMEM((2, _CHUNK, _P), jnp.float32),
            pltpu.VMEM((_NSLOT, _CHUNK, _P), jnp.float32),
            pltpu.SemaphoreType.DMA((2,)),
            pltpu.SemaphoreType.DMA((_NSLOT,)),
            pltpu.SemaphoreType.DMA((_NSLOT,)),
        ],
    )(_sc_kernel)
    out2d = run(x2d, t2d)
    return jnp.swapaxes(out2d.reshape(B, D, P), 1, 2)
